# baseline (speedup 1.0000x reference)
import functools

import jax
import jax.numpy as jnp
from jax import lax
from jax.experimental import pallas as pl
from jax.experimental.pallas import tpu as pltpu
from jax.experimental.pallas import tpu_sc as plsc

_D = 768
_K = 8
_NEG = -1e30
_BQ = 256
_GCH = 32
_NBUF = 4


def _embed_gather_sc(idx_flat, embed):
    info = plsc.get_sparse_core_info()
    nw = info.num_cores * info.num_subcores
    bs = idx_flat.shape[0]
    rows_per_w = bs // nw
    nch = rows_per_w // _GCH
    mesh = plsc.VectorSubcoreMesh(core_axis_name="c", subcore_axis_name="s")

    @functools.partial(
        pl.kernel,
        out_type=jax.ShapeDtypeStruct((bs, _D), jnp.float32),
        mesh=mesh,
        scratch_types=(
            [pltpu.VMEM((rows_per_w,), jnp.int32)]
            + [pltpu.VMEM((_GCH, _D), jnp.float32) for _ in range(_NBUF)]
            + [pltpu.SemaphoreType.DMA for _ in range(2 * _NBUF)]
        ),
    )
    def gather_k(idx_hbm, table_hbm, out_hbm, idx_v, *rest):
        bufs = rest[:_NBUF]
        gsem = rest[_NBUF:2 * _NBUF]
        osem = rest[2 * _NBUF:]
        wid = lax.axis_index("s") * info.num_cores + lax.axis_index("c")
        base = wid * rows_per_w
        pltpu.sync_copy(idx_hbm.at[pl.ds(base, rows_per_w)], idx_v)
        gcp = [None] * nch
        ocp = [None] * nch
        for ci in range(min(_NBUF, nch)):
            gcp[ci] = pltpu.async_copy(
                table_hbm.at[idx_v.at[pl.ds(ci * _GCH, _GCH)]],
                bufs[ci], gsem[ci])
        for ci in range(nch):
            bi = ci % _NBUF
            gcp[ci].wait()
            ocp[ci] = pltpu.async_copy(
                bufs[bi], out_hbm.at[pl.ds(base + ci * _GCH, _GCH)], osem[bi])
            nxt = ci + _NBUF
            if nxt < nch:
                ocp[ci].wait()
                gcp[nxt] = pltpu.async_copy(
                    table_hbm.at[idx_v.at[pl.ds(nxt * _GCH, _GCH)]],
                    bufs[bi], gsem[bi])
        for ci in range(max(0, nch - _NBUF), nch):
            ocp[ci].wait()

    return gather_k(idx_flat, embed)


def _layer_core(xq, xf, wr, wv, gl):
    scale = 1.0 / (_D ** 0.5)
    xqw = xq * wr[None, :]
    scores = lax.dot_general(
        xqw, xf, (((1,), (1,)), ((), ())),
        preferred_element_type=jnp.float32) * scale
    bq, s = scores.shape
    nch = s // 128
    chunks = [scores[:, c * 128:(c + 1) * 128] for c in range(nch)]
    pm = chunks[0]
    for c in range(1, nch):
        pm = jnp.maximum(pm, chunks[c])
    thresh = jnp.full((bq, 1), -jnp.inf, jnp.float32)
    m1 = None
    for i in range(_K - 1):
        m = jnp.max(pm, axis=1, keepdims=True)
        if i == 0:
            m1 = m
        cntv = None
        pm = None
        for c in range(nch):
            wc = chunks[c]
            ge = wc >= m
            contrib = jnp.where(ge, 1.0, 0.0)
            cntv = contrib if cntv is None else cntv + contrib
            masked = jnp.where(ge, _NEG, wc)
            pm = masked if pm is None else jnp.maximum(pm, masked)
        cnt = jnp.sum(cntv, axis=1, keepdims=True)
        thresh = jnp.maximum(thresh, jnp.where(cnt >= _K, m, -jnp.inf))
    thresh = jnp.maximum(thresh, jnp.max(pm, axis=1, keepdims=True))
    e = jnp.where(scores >= thresh, jnp.exp(scores - m1), 0.0)
    den = jnp.sum(e, axis=1, keepdims=True)
    mix = lax.dot_general(
        e.astype(jnp.bfloat16), xf.astype(jnp.bfloat16),
        (((1,), (0,)), ((), ())),
        preferred_element_type=jnp.float32)
    upd = lax.dot_general(
        (mix / den).astype(jnp.bfloat16), wv.astype(jnp.bfloat16),
        (((1,), (0,)), ((), ())),
        preferred_element_type=jnp.float32)
    y = xq + upd
    r = lax.rsqrt(jnp.mean(y * y, axis=1, keepdims=True) + 1e-6)
    return y * r * gl[None, :]


def _layer_body(xq_ref, xf_ref, wr_ref, wv_ref, g_ref, out_ref):
    out_ref[0] = _layer_core(
        xq_ref[0], xf_ref[0], wr_ref[0], wv_ref[...], g_ref[0])


def _layer(x, wr, wv, gl):
    b, s, d = x.shape
    return pl.pallas_call(
        _layer_body,
        grid=(b, s // _BQ),
        in_specs=[
            pl.BlockSpec((1, _BQ, d), lambda i, j: (i, j, 0)),
            pl.BlockSpec((1, s, d), lambda i, j: (i, 0, 0)),
            pl.BlockSpec((1, d), lambda i, j: (0, 0)),
            pl.BlockSpec((d, d), lambda i, j: (0, 0)),
            pl.BlockSpec((1, d), lambda i, j: (0, 0)),
        ],
        out_specs=pl.BlockSpec((1, _BQ, d), lambda i, j: (i, j, 0)),
        out_shape=jax.ShapeDtypeStruct((b, s, d), jnp.float32),
    )(x, x, wr.reshape(1, d), wv, gl.reshape(1, d))


def _layer_logits_body(xq_ref, xf_ref, wr_ref, wv_ref, g_ref, emb_ref,
                       out_ref):
    out = _layer_core(
        xq_ref[0], xf_ref[0], wr_ref[0], wv_ref[...], g_ref[0])
    out_ref[0] = lax.dot_general(
        out.astype(jnp.bfloat16), emb_ref[...], (((1,), (1,)), ((), ())),
        preferred_element_type=jnp.float32)


def _layer_logits(x, wr, wv, gl, emb_bf):
    b, s, d = x.shape
    v = emb_bf.shape[0]
    return pl.pallas_call(
        _layer_logits_body,
        grid=(b, s // _BQ),
        in_specs=[
            pl.BlockSpec((1, _BQ, d), lambda i, j: (i, j, 0)),
            pl.BlockSpec((1, s, d), lambda i, j: (i, 0, 0)),
            pl.BlockSpec((1, d), lambda i, j: (0, 0)),
            pl.BlockSpec((d, d), lambda i, j: (0, 0)),
            pl.BlockSpec((1, d), lambda i, j: (0, 0)),
            pl.BlockSpec((v, d), lambda i, j: (0, 0)),
        ],
        out_specs=pl.BlockSpec((1, _BQ, v), lambda i, j: (i, j, 0)),
        out_shape=jax.ShapeDtypeStruct((b, s, v), jnp.float32),
    )(x, x, wr.reshape(1, d), wv, gl.reshape(1, d), emb_bf)


def kernel(tokens, embed, w_route, W_val, g):
    b, s = tokens.shape
    v, d = embed.shape
    nl = w_route.shape[0]
    last = nl - 1
    emb_bf = embed.astype(jnp.bfloat16)
    outs = []
    for i in range(b):
        xi = _embed_gather_sc(tokens[i].astype(jnp.int32), embed)
        xi = xi.reshape(1, s, d)
        for l in range(nl - 1):
            xi = _layer(xi, w_route[l], W_val[l], g[l])
        outs.append(_layer_logits(xi, w_route[last], W_val[last], g[last],
                                  emb_bf))
    return jnp.concatenate(outs, axis=0)


# speedup vs baseline: 1.3919x; 1.3919x over previous
import functools

import jax
import jax.numpy as jnp
from jax import lax
from jax.experimental import pallas as pl
from jax.experimental.pallas import tpu as pltpu
from jax.experimental.pallas import tpu_sc as plsc

_D = 768
_K = 8
_NEG = -1e30
_BQ = 256
_GCH = 32
_NBUF = 4


def _embed_gather_sc(idx_flat, embed):
    info = plsc.get_sparse_core_info()
    nw = info.num_cores * info.num_subcores
    bs = idx_flat.shape[0]
    rows_per_w = bs // nw
    nch = rows_per_w // _GCH
    mesh = plsc.VectorSubcoreMesh(core_axis_name="c", subcore_axis_name="s")

    @functools.partial(
        pl.kernel,
        out_type=jax.ShapeDtypeStruct((bs, _D), jnp.float32),
        mesh=mesh,
        scratch_types=(
            [pltpu.VMEM((rows_per_w,), jnp.int32)]
            + [pltpu.VMEM((_GCH, _D), jnp.float32) for _ in range(_NBUF)]
            + [pltpu.SemaphoreType.DMA for _ in range(2 * _NBUF)]
        ),
    )
    def gather_k(idx_hbm, table_hbm, out_hbm, idx_v, *rest):
        bufs = rest[:_NBUF]
        gsem = rest[_NBUF:2 * _NBUF]
        osem = rest[2 * _NBUF:]
        wid = lax.axis_index("s") * info.num_cores + lax.axis_index("c")
        base = wid * rows_per_w
        pltpu.sync_copy(idx_hbm.at[pl.ds(base, rows_per_w)], idx_v)
        gcp = [None] * nch
        ocp = [None] * nch
        for ci in range(min(_NBUF, nch)):
            gcp[ci] = pltpu.async_copy(
                table_hbm.at[idx_v.at[pl.ds(ci * _GCH, _GCH)]],
                bufs[ci], gsem[ci])
        for ci in range(nch):
            bi = ci % _NBUF
            gcp[ci].wait()
            ocp[ci] = pltpu.async_copy(
                bufs[bi], out_hbm.at[pl.ds(base + ci * _GCH, _GCH)], osem[bi])
            nxt = ci + _NBUF
            if nxt < nch:
                ocp[ci].wait()
                gcp[nxt] = pltpu.async_copy(
                    table_hbm.at[idx_v.at[pl.ds(nxt * _GCH, _GCH)]],
                    bufs[bi], gsem[bi])
        for ci in range(max(0, nch - _NBUF), nch):
            ocp[ci].wait()

    return gather_k(idx_flat, embed)


def _layer_core(xq, xf, wr, wv, gl):
    scale = 1.0 / (_D ** 0.5)
    xqw = xq * wr[None, :]
    scores = lax.dot_general(
        xqw, xf, (((1,), (1,)), ((), ())),
        preferred_element_type=jnp.float32) * scale
    bq, s = scores.shape
    nch = s // 128
    chunks = [scores[:, c * 128:(c + 1) * 128] for c in range(nch)]
    pm = chunks[0]
    for c in range(1, nch):
        pm = jnp.maximum(pm, chunks[c])
    thresh = jnp.full((bq, 1), -jnp.inf, jnp.float32)
    m1 = None
    for i in range(_K - 1):
        m = jnp.max(pm, axis=1, keepdims=True)
        if i == 0:
            m1 = m
        cntv = None
        pm = None
        for c in range(nch):
            wc = chunks[c]
            ge = wc >= m
            contrib = jnp.where(ge, 1.0, 0.0)
            cntv = contrib if cntv is None else cntv + contrib
            masked = jnp.where(ge, _NEG, wc)
            pm = masked if pm is None else jnp.maximum(pm, masked)
        cnt = jnp.sum(cntv, axis=1, keepdims=True)
        thresh = jnp.maximum(thresh, jnp.where(cnt >= _K, m, -jnp.inf))
    thresh = jnp.maximum(thresh, jnp.max(pm, axis=1, keepdims=True))
    e = jnp.where(scores >= thresh, jnp.exp(scores - m1), 0.0)
    den = jnp.sum(e, axis=1, keepdims=True)
    mix = lax.dot_general(
        e.astype(jnp.bfloat16), xf.astype(jnp.bfloat16),
        (((1,), (0,)), ((), ())),
        preferred_element_type=jnp.float32)
    upd = lax.dot_general(
        (mix / den).astype(jnp.bfloat16), wv.astype(jnp.bfloat16),
        (((1,), (0,)), ((), ())),
        preferred_element_type=jnp.float32)
    y = xq + upd
    r = lax.rsqrt(jnp.mean(y * y, axis=1, keepdims=True) + 1e-6)
    return y * r * gl[None, :]


def _layer_body(xq_ref, xf_ref, wr_ref, wv_ref, g_ref, out_ref):
    out_ref[0] = _layer_core(
        xq_ref[0], xf_ref[0], wr_ref[0], wv_ref[...], g_ref[0])


def _layer(x, wr, wv, gl, bq=_BQ):
    b, s, d = x.shape
    return pl.pallas_call(
        _layer_body,
        grid=(b, s // bq),
        in_specs=[
            pl.BlockSpec((1, bq, d), lambda i, j: (i, j, 0)),
            pl.BlockSpec((1, s, d), lambda i, j: (i, 0, 0)),
            pl.BlockSpec((1, d), lambda i, j: (0, 0)),
            pl.BlockSpec((d, d), lambda i, j: (0, 0)),
            pl.BlockSpec((1, d), lambda i, j: (0, 0)),
        ],
        out_specs=pl.BlockSpec((1, bq, d), lambda i, j: (i, j, 0)),
        out_shape=jax.ShapeDtypeStruct((b, s, d), jnp.float32),
    )(x, x, wr.reshape(1, d), wv, gl.reshape(1, d))


def _layer_logits_body(xq_ref, xf_ref, wr_ref, wv_ref, g_ref, emb_ref,
                       out_ref):
    out = _layer_core(
        xq_ref[0], xf_ref[0], wr_ref[0], wv_ref[...], g_ref[0])
    out_ref[0] = lax.dot_general(
        out.astype(jnp.bfloat16), emb_ref[...], (((1,), (1,)), ((), ())),
        preferred_element_type=jnp.float32)


def _layer_logits(x, wr, wv, gl, emb_bf):
    b, s, d = x.shape
    v = emb_bf.shape[0]
    return pl.pallas_call(
        _layer_logits_body,
        grid=(b, s // _BQ),
        in_specs=[
            pl.BlockSpec((1, _BQ, d), lambda i, j: (i, j, 0)),
            pl.BlockSpec((1, s, d), lambda i, j: (i, 0, 0)),
            pl.BlockSpec((1, d), lambda i, j: (0, 0)),
            pl.BlockSpec((d, d), lambda i, j: (0, 0)),
            pl.BlockSpec((1, d), lambda i, j: (0, 0)),
            pl.BlockSpec((v, d), lambda i, j: (0, 0)),
        ],
        out_specs=pl.BlockSpec((1, _BQ, v), lambda i, j: (i, j, 0)),
        out_shape=jax.ShapeDtypeStruct((b, s, v), jnp.float32),
    )(x, x, wr.reshape(1, d), wv, gl.reshape(1, d), emb_bf)


def kernel(tokens, embed, w_route, W_val, g):
    b, s = tokens.shape
    v, d = embed.shape
    x = _embed_gather_sc(tokens.reshape(-1).astype(jnp.int32), embed)
    x = x.reshape(b, s, d)
    nl = w_route.shape[0]
    for l in range(nl - 1):
        x = _layer(x, w_route[l], W_val[l], g[l], bq=512)
    last = nl - 1
    return _layer_logits(x, w_route[last], W_val[last], g[last],
                         embed.astype(jnp.bfloat16))


# speedup vs baseline: 1.4023x; 1.0075x over previous
import functools

import jax
import jax.numpy as jnp
from jax import lax
from jax.experimental import pallas as pl
from jax.experimental.pallas import tpu as pltpu
from jax.experimental.pallas import tpu_sc as plsc

_D = 768
_K = 8
_NEG = -1e30
_BQ = 256
_GCH = 32
_NBUF = 4


def _embed_gather_sc(idx_flat, embed):
    info = plsc.get_sparse_core_info()
    nw = info.num_cores * info.num_subcores
    bs = idx_flat.shape[0]
    rows_per_w = bs // nw
    nch = rows_per_w // _GCH
    mesh = plsc.VectorSubcoreMesh(core_axis_name="c", subcore_axis_name="s")

    @functools.partial(
        pl.kernel,
        out_type=jax.ShapeDtypeStruct((bs, _D), jnp.float32),
        mesh=mesh,
        scratch_types=(
            [pltpu.VMEM((rows_per_w,), jnp.int32)]
            + [pltpu.VMEM((_GCH, _D), jnp.float32) for _ in range(_NBUF)]
            + [pltpu.SemaphoreType.DMA for _ in range(2 * _NBUF)]
        ),
    )
    def gather_k(idx_hbm, table_hbm, out_hbm, idx_v, *rest):
        bufs = rest[:_NBUF]
        gsem = rest[_NBUF:2 * _NBUF]
        osem = rest[2 * _NBUF:]
        wid = lax.axis_index("s") * info.num_cores + lax.axis_index("c")
        base = wid * rows_per_w
        pltpu.sync_copy(idx_hbm.at[pl.ds(base, rows_per_w)], idx_v)
        gcp = [None] * nch
        ocp = [None] * nch
        for ci in range(min(_NBUF, nch)):
            gcp[ci] = pltpu.async_copy(
                table_hbm.at[idx_v.at[pl.ds(ci * _GCH, _GCH)]],
                bufs[ci], gsem[ci])
        for ci in range(nch):
            bi = ci % _NBUF
            gcp[ci].wait()
            ocp[ci] = pltpu.async_copy(
                bufs[bi], out_hbm.at[pl.ds(base + ci * _GCH, _GCH)], osem[bi])
            nxt = ci + _NBUF
            if nxt < nch:
                ocp[ci].wait()
                gcp[nxt] = pltpu.async_copy(
                    table_hbm.at[idx_v.at[pl.ds(nxt * _GCH, _GCH)]],
                    bufs[bi], gsem[bi])
        for ci in range(max(0, nch - _NBUF), nch):
            ocp[ci].wait()

    return gather_k(idx_flat, embed)


def _layer_core(xq, xf, wr, wv, gl):
    scale = 1.0 / (_D ** 0.5)
    xqw = xq * wr[None, :]
    scores = lax.dot_general(
        xqw, xf, (((1,), (1,)), ((), ())),
        preferred_element_type=jnp.float32) * scale
    bq, s = scores.shape
    nch = s // 128
    chunks = [scores[:, c * 128:(c + 1) * 128] for c in range(nch)]
    pm = chunks[0]
    for c in range(1, nch):
        pm = jnp.maximum(pm, chunks[c])
    thresh = jnp.full((bq, 1), -jnp.inf, jnp.float32)
    m1 = None
    for i in range(_K - 1):
        m = jnp.max(pm, axis=1, keepdims=True)
        if i == 0:
            m1 = m
        cntv = None
        pm = None
        for c in range(nch):
            wc = chunks[c]
            ge = wc >= m
            contrib = jnp.where(ge, 1.0, 0.0)
            cntv = contrib if cntv is None else cntv + contrib
            masked = jnp.where(ge, _NEG, wc)
            pm = masked if pm is None else jnp.maximum(pm, masked)
        cnt = jnp.sum(cntv, axis=1, keepdims=True)
        thresh = jnp.maximum(thresh, jnp.where(cnt >= _K, m, -jnp.inf))
    thresh = jnp.maximum(thresh, jnp.max(pm, axis=1, keepdims=True))
    e = jnp.where(scores >= thresh, jnp.exp(scores - m1), 0.0)
    den = jnp.sum(e, axis=1, keepdims=True)
    mix = lax.dot_general(
        e.astype(jnp.bfloat16), xf.astype(jnp.bfloat16),
        (((1,), (0,)), ((), ())),
        preferred_element_type=jnp.float32)
    upd = lax.dot_general(
        (mix / den).astype(jnp.bfloat16), wv.astype(jnp.bfloat16),
        (((1,), (0,)), ((), ())),
        preferred_element_type=jnp.float32)
    y = xq + upd
    r = lax.rsqrt(jnp.mean(y * y, axis=1, keepdims=True) + 1e-6)
    return y * r * gl[None, :]


def _layer_body(xq_ref, xf_ref, wr_ref, wv_ref, g_ref, out_ref):
    out_ref[0] = _layer_core(
        xq_ref[0], xf_ref[0], wr_ref[0], wv_ref[...], g_ref[0])


def _layer(x, wr, wv, gl, bq=_BQ):
    b, s, d = x.shape
    return pl.pallas_call(
        _layer_body,
        grid=(b, s // bq),
        in_specs=[
            pl.BlockSpec((1, bq, d), lambda i, j: (i, j, 0)),
            pl.BlockSpec((1, s, d), lambda i, j: (i, 0, 0)),
            pl.BlockSpec((1, d), lambda i, j: (0, 0)),
            pl.BlockSpec((d, d), lambda i, j: (0, 0)),
            pl.BlockSpec((1, d), lambda i, j: (0, 0)),
        ],
        out_specs=pl.BlockSpec((1, bq, d), lambda i, j: (i, j, 0)),
        out_shape=jax.ShapeDtypeStruct((b, s, d), jnp.float32),
    )(x, x, wr.reshape(1, d), wv, gl.reshape(1, d))


def _layer_logits_body(xq_ref, xf_ref, wr_ref, wv_ref, g_ref, emb_ref,
                       out_ref):
    out = _layer_core(
        xq_ref[0], xf_ref[0], wr_ref[0], wv_ref[...], g_ref[0])
    out_ref[0] = lax.dot_general(
        out.astype(jnp.bfloat16), emb_ref[...], (((1,), (1,)), ((), ())),
        preferred_element_type=jnp.float32)


def _layer_logits(x, wr, wv, gl, emb_bf):
    b, s, d = x.shape
    v = emb_bf.shape[0]
    return pl.pallas_call(
        _layer_logits_body,
        grid=(b, s // _BQ),
        in_specs=[
            pl.BlockSpec((1, _BQ, d), lambda i, j: (i, j, 0)),
            pl.BlockSpec((1, s, d), lambda i, j: (i, 0, 0)),
            pl.BlockSpec((1, d), lambda i, j: (0, 0)),
            pl.BlockSpec((d, d), lambda i, j: (0, 0)),
            pl.BlockSpec((1, d), lambda i, j: (0, 0)),
            pl.BlockSpec((v, d), lambda i, j: (0, 0)),
        ],
        out_specs=pl.BlockSpec((1, _BQ, v), lambda i, j: (i, j, 0)),
        out_shape=jax.ShapeDtypeStruct((b, s, v), jnp.float32),
    )(x, x, wr.reshape(1, d), wv, gl.reshape(1, d), emb_bf)


def kernel(tokens, embed, w_route, W_val, g):
    b, s = tokens.shape
    v, d = embed.shape
    x = _embed_gather_sc(tokens.reshape(-1).astype(jnp.int32), embed)
    x = x.reshape(b, s, d)
    nl = w_route.shape[0]
    for l in range(nl - 1):
        x = _layer(x, w_route[l], W_val[l], g[l], bq=1024)
    last = nl - 1
    return _layer_logits(x, w_route[last], W_val[last], g[last],
                         embed.astype(jnp.bfloat16))
